# baseline (speedup 1.0000x reference)
import jax
import jax.numpy as jnp
from jax import lax
from jax.experimental import pallas as pl
from jax.experimental.pallas import tpu as pltpu
from jax.experimental.pallas import tpu_sc as plsc

NUM_CORES = 2
NUM_SUBCORES = 16
NUM_LANES = 16
NW = NUM_CORES * NUM_SUBCORES

NUM_ROWS = 1000000
BATCH = 16384
EMB_DIM = 32
B_PER_W = BATCH // NW
IDX_CHUNK = 128
N_CHUNKS = B_PER_W // IDX_CHUNK
N_VECS = B_PER_W // NUM_LANES

ALIGNED_ROWS = 999936
TAIL_ROWS = NUM_ROWS - ALIGNED_ROWS
TAIL_SIZE = EMB_DIM * TAIL_ROWS
FLAT_BULK = EMB_DIM * ALIGNED_ROWS
DETILE_Q = ALIGNED_ROWS // 4


def _mf_body(user_ref, item_ref, ub_ref, ib_ref, utail_ref, itail_ref,
             out_ref, idx_u, idx_i, off, elems_u, elems_i, tail_u, tail_i,
             out_v, sem):
  wid = lax.axis_index("s") * NUM_CORES + lax.axis_index("c")
  base = wid * B_PER_W

  pltpu.sync_copy(user_ref.at[pl.ds(base, B_PER_W)], idx_u)
  pltpu.sync_copy(item_ref.at[pl.ds(base, B_PER_W)], idx_i)
  pltpu.sync_copy(utail_ref, tail_u)
  pltpu.sync_copy(itail_ref, tail_i)

  def for_each_j(j, table_ref, idx, elems):
    def build(v, carry):
      sl = pl.ds(v * NUM_LANES, NUM_LANES)
      r = jnp.minimum(idx[sl], ALIGNED_ROWS - 1)
      off[sl] = r + j * ALIGNED_ROWS
      return carry
    lax.fori_loop(0, N_VECS, build, 0)
    for c in range(N_CHUNKS):
      sl = pl.ds(c * IDX_CHUNK, IDX_CHUNK)
      pltpu.async_copy(table_ref.at[off.at[sl]], elems.at[j, sl], sem)
    for c in range(N_CHUNKS):
      sl = pl.ds(c * IDX_CHUNK, IDX_CHUNK)
      pltpu.make_async_copy(table_ref.at[off.at[sl]], elems.at[j, sl],
                            sem).wait()

  for j in range(EMB_DIM):
    for_each_j(j, ub_ref, idx_u, elems_u)
    for_each_j(j, ib_ref, idx_i, elems_i)

  def group(m, carry):
    sl = pl.ds(m * NUM_LANES, NUM_LANES)
    ru = idx_u[sl]
    ri = idx_i[sl]
    mu = ru >= ALIGNED_ROWS
    mi = ri >= ALIGNED_ROWS
    tbu = jnp.clip((ru - ALIGNED_ROWS) * EMB_DIM, 0, TAIL_SIZE - EMB_DIM)
    tbi = jnp.clip((ri - ALIGNED_ROWS) * EMB_DIM, 0, TAIL_SIZE - EMB_DIM)
    acc = jnp.zeros((NUM_LANES,), jnp.float32)
    for j in range(EMB_DIM):
      jv = jnp.full((NUM_LANES,), j, jnp.int32)
      u = jnp.where(mu, plsc.load_gather(tail_u, [tbu + jv]),
                    elems_u[j, sl])
      v = jnp.where(mi, plsc.load_gather(tail_i, [tbi + jv]),
                    elems_i[j, sl])
      acc = acc + u * v
    out_v[sl] = 1.0 / (1.0 + jnp.exp(-acc))
    return carry

  lax.fori_loop(0, N_VECS, group, 0)

  pltpu.sync_copy(out_v, out_ref.at[pl.ds(base, B_PER_W)])


def _detile_body(ut_ref, it_ref, uflat_ref, iflat_ref, sem):
  def copies():
    for table, flat in ((ut_ref, uflat_ref), (it_ref, iflat_ref)):
      fview = flat.reshape(EMB_DIM, ALIGNED_ROWS)
      for g in range(EMB_DIM // 8):
        for q in range(4):
          sl = pl.ds(q * DETILE_Q, DETILE_Q)
          yield (table.at[pl.ds(8 * g, 8), sl],
                 fview.at[pl.ds(8 * g, 8), sl])

  for src, dst in copies():
    pltpu.make_async_copy(src, dst, sem).start()
  for src, dst in copies():
    pltpu.make_async_copy(src, dst, sem).wait()


def _detile(ut, it):
  return pl.pallas_call(
      _detile_body,
      in_specs=[pl.BlockSpec(memory_space=pl.ANY)] * 2,
      out_specs=[pl.BlockSpec(memory_space=pl.ANY)] * 2,
      out_shape=[jax.ShapeDtypeStruct((FLAT_BULK // 128, 128),
                                      jnp.float32)] * 2,
      scratch_shapes=[pltpu.SemaphoreType.DMA],
  )(ut, it)


@jax.jit
def kernel(user, item, user_table, item_table):
  utail = user_table[ALIGNED_ROWS:].reshape(-1)
  itail = item_table[ALIGNED_ROWS:].reshape(-1)
  ub, ib = _detile(user_table.T, item_table.T)
  ub = ub.reshape(FLAT_BULK)
  ib = ib.reshape(FLAT_BULK)
  mesh = plsc.VectorSubcoreMesh(core_axis_name="c", subcore_axis_name="s")
  run = pl.kernel(
      _mf_body,
      out_type=jax.ShapeDtypeStruct((BATCH,), jnp.float32),
      mesh=mesh,
      compiler_params=pltpu.CompilerParams(
          needs_layout_passes=False,
          use_tc_tiling_on_sc=False,
      ),
      scratch_types=[
          pltpu.VMEM((B_PER_W,), jnp.int32),
          pltpu.VMEM((B_PER_W,), jnp.int32),
          pltpu.VMEM((B_PER_W,), jnp.int32),
          pltpu.VMEM((EMB_DIM, B_PER_W), jnp.float32),
          pltpu.VMEM((EMB_DIM, B_PER_W), jnp.float32),
          pltpu.VMEM((TAIL_SIZE,), jnp.float32),
          pltpu.VMEM((TAIL_SIZE,), jnp.float32),
          pltpu.VMEM((B_PER_W,), jnp.float32),
          pltpu.SemaphoreType.DMA,
      ],
  )
  return run(user, item, ub, ib, utail, itail)


# speedup vs baseline: 1.5269x; 1.5269x over previous
import jax
import jax.numpy as jnp
from jax import lax
from jax.experimental import pallas as pl
from jax.experimental.pallas import tpu as pltpu
from jax.experimental.pallas import tpu_sc as plsc

NUM_CORES = 2
NUM_SUBCORES = 16
NUM_LANES = 16
NW = NUM_CORES * NUM_SUBCORES

NUM_ROWS = 1000000
BATCH = 16384
EMB_DIM = 32
B_PER_W = BATCH // NW
IDX_CHUNK = 128
N_CHUNKS = B_PER_W // IDX_CHUNK
N_VECS = B_PER_W // NUM_LANES


def _mf_body(user_ref, item_ref, ut_ref, it_ref, out_ref,
             idx_u, idx_i, elems_u, elems_i, out_v, sem):
  wid = lax.axis_index("s") * NUM_CORES + lax.axis_index("c")
  base = wid * B_PER_W

  pltpu.sync_copy(user_ref.at[pl.ds(base, B_PER_W)], idx_u)
  pltpu.sync_copy(item_ref.at[pl.ds(base, B_PER_W)], idx_i)

  def for_each_j(j, table_ref, idx, elems):
    for c in range(N_CHUNKS):
      sl = pl.ds(c * IDX_CHUNK, IDX_CHUNK)
      pltpu.async_copy(table_ref.at[j].at[idx.at[sl]], elems.at[j, sl], sem)
    for c in range(N_CHUNKS):
      sl = pl.ds(c * IDX_CHUNK, IDX_CHUNK)
      pltpu.make_async_copy(table_ref.at[j].at[idx.at[sl]],
                            elems.at[j, sl], sem).wait()

  for j in range(EMB_DIM):
    for_each_j(j, ut_ref, idx_u, elems_u)
    for_each_j(j, it_ref, idx_i, elems_i)

  def group(m, carry):
    sl = pl.ds(m * NUM_LANES, NUM_LANES)
    acc = elems_u[0, sl] * elems_i[0, sl]
    for j in range(1, EMB_DIM):
      acc = acc + elems_u[j, sl] * elems_i[j, sl]
    out_v[sl] = 1.0 / (1.0 + jnp.exp(-acc))
    return carry

  lax.fori_loop(0, N_VECS, group, 0)

  pltpu.sync_copy(out_v, out_ref.at[pl.ds(base, B_PER_W)])


@jax.jit
def kernel(user, item, user_table, item_table):
  mesh = plsc.VectorSubcoreMesh(core_axis_name="c", subcore_axis_name="s")
  run = pl.kernel(
      _mf_body,
      out_type=jax.ShapeDtypeStruct((BATCH,), jnp.float32),
      mesh=mesh,
      compiler_params=pltpu.CompilerParams(
          needs_layout_passes=False,
          use_tc_tiling_on_sc=False,
      ),
      scratch_types=[
          pltpu.VMEM((B_PER_W,), jnp.int32),
          pltpu.VMEM((B_PER_W,), jnp.int32),
          pltpu.VMEM((EMB_DIM, B_PER_W), jnp.float32),
          pltpu.VMEM((EMB_DIM, B_PER_W), jnp.float32),
          pltpu.VMEM((B_PER_W,), jnp.float32),
          pltpu.SemaphoreType.DMA,
      ],
  )
  return run(user, item, user_table.T, item_table.T)


# speedup vs baseline: 8.6764x; 5.6824x over previous
import functools

import jax
import jax.numpy as jnp
from jax import lax
from jax.experimental import pallas as pl
from jax.experimental.pallas import tpu as pltpu
from jax.experimental.pallas import tpu_sc as plsc

NUM_CORES = 2
NUM_SUBCORES = 16
NUM_LANES = 16
NW = NUM_CORES * NUM_SUBCORES

BATCH = 16384
EMB_DIM = 32
B_PER_W = BATCH // NW
IDX_CHUNK = 128
N_CHUNKS = B_PER_W // IDX_CHUNK


def _mf_body(user_ref, item_ref, user_table, item_table, out_ref,
             idx_u, idx_i, rows_u, rows_i, out_v, sem):
  wid = lax.axis_index("s") * NUM_CORES + lax.axis_index("c")
  base = wid * B_PER_W

  pltpu.sync_copy(user_ref.at[pl.ds(base, B_PER_W)], idx_u)
  pltpu.sync_copy(item_ref.at[pl.ds(base, B_PER_W)], idx_i)

  for j in range(N_CHUNKS):
    sl = pl.ds(j * IDX_CHUNK, IDX_CHUNK)
    pltpu.async_copy(user_table.at[idx_u.at[sl]], rows_u.at[sl], sem)
    pltpu.async_copy(item_table.at[idx_i.at[sl]], rows_i.at[sl], sem)
  for j in range(N_CHUNKS):
    sl = pl.ds(j * IDX_CHUNK, IDX_CHUNK)
    pltpu.make_async_copy(user_table.at[idx_u.at[sl]], rows_u.at[sl], sem).wait()
    pltpu.make_async_copy(item_table.at[idx_i.at[sl]], rows_i.at[sl], sem).wait()

  lane = lax.iota(jnp.int32, NUM_LANES)

  def group(g, carry):
    rows = g * NUM_LANES + lane
    acc = jnp.zeros((NUM_LANES,), jnp.float32)
    for d in range(EMB_DIM):
      col = jnp.full((NUM_LANES,), d, jnp.int32)
      u = plsc.load_gather(rows_u, [rows, col])
      v = plsc.load_gather(rows_i, [rows, col])
      acc = acc + u * v
    sig = 1.0 / (1.0 + jnp.exp(-acc))
    out_v[pl.ds(g * NUM_LANES, NUM_LANES)] = sig
    return carry

  lax.fori_loop(0, B_PER_W // NUM_LANES, group, 0)

  pltpu.sync_copy(out_v, out_ref.at[pl.ds(base, B_PER_W)])


@jax.jit
def kernel(user, item, user_table, item_table):
  mesh = plsc.VectorSubcoreMesh(core_axis_name="c", subcore_axis_name="s")
  run = pl.kernel(
      _mf_body,
      out_type=jax.ShapeDtypeStruct((BATCH,), jnp.float32),
      mesh=mesh,
      compiler_params=pltpu.CompilerParams(
          needs_layout_passes=False,
          use_tc_tiling_on_sc=False,
      ),
      scratch_types=[
          pltpu.VMEM((B_PER_W,), jnp.int32),
          pltpu.VMEM((B_PER_W,), jnp.int32),
          pltpu.VMEM((B_PER_W, EMB_DIM), jnp.float32),
          pltpu.VMEM((B_PER_W, EMB_DIM), jnp.float32),
          pltpu.VMEM((B_PER_W,), jnp.float32),
          pltpu.SemaphoreType.DMA,
      ],
  )
  return run(user, item, user_table, item_table)


# speedup vs baseline: 8.6905x; 1.0016x over previous
import jax
import jax.numpy as jnp
from jax import lax
from jax.experimental import pallas as pl
from jax.experimental.pallas import tpu as pltpu
from jax.experimental.pallas import tpu_sc as plsc

NUM_CORES = 2
NUM_SUBCORES = 16
NUM_LANES = 16
NW = NUM_CORES * NUM_SUBCORES

BATCH = 16384
EMB_DIM = 32
B_PER_W = BATCH // NW
IDX_CHUNK = 128
N_CHUNKS = B_PER_W // IDX_CHUNK


def _mf_body(user_ref, item_ref, user_table, item_table, out_ref,
             idx_u, idx_i, rows_u, rows_i, out_v, sem):
  wid = lax.axis_index("s") * NUM_CORES + lax.axis_index("c")
  base = wid * B_PER_W

  pltpu.sync_copy(user_ref.at[pl.ds(base, B_PER_W)], idx_u)
  pltpu.sync_copy(item_ref.at[pl.ds(base, B_PER_W)], idx_i)

  for j in range(N_CHUNKS):
    sl = pl.ds(j * IDX_CHUNK, IDX_CHUNK)
    pltpu.async_copy(user_table.at[idx_u.at[sl]], rows_u.at[sl], sem)
    pltpu.async_copy(item_table.at[idx_i.at[sl]], rows_i.at[sl], sem)
  for j in range(N_CHUNKS):
    sl = pl.ds(j * IDX_CHUNK, IDX_CHUNK)
    pltpu.make_async_copy(user_table.at[idx_u.at[sl]], rows_u.at[sl], sem).wait()
    pltpu.make_async_copy(item_table.at[idx_i.at[sl]], rows_i.at[sl], sem).wait()

  lane = lax.iota(jnp.int32, NUM_LANES)

  def group(g, carry):
    rows = g * NUM_LANES + lane
    acc = jnp.zeros((NUM_LANES,), jnp.float32)
    for d in range(EMB_DIM):
      col = jnp.full((NUM_LANES,), d, jnp.int32)
      u = plsc.load_gather(rows_u, [rows, col])
      v = plsc.load_gather(rows_i, [rows, col])
      acc = acc + u * v
    sig = 1.0 / (1.0 + jnp.exp(-acc))
    out_v[pl.ds(g * NUM_LANES, NUM_LANES)] = sig
    return carry

  lax.fori_loop(0, B_PER_W // NUM_LANES, group, 0)

  pltpu.sync_copy(out_v, out_ref.at[pl.ds(base, B_PER_W)])


@jax.jit
def kernel(user, item, user_table, item_table):
  mesh = plsc.VectorSubcoreMesh(core_axis_name="c", subcore_axis_name="s")
  run = pl.kernel(
      _mf_body,
      out_type=jax.ShapeDtypeStruct((BATCH,), jnp.float32),
      mesh=mesh,
      compiler_params=pltpu.CompilerParams(
          needs_layout_passes=False,
          use_tc_tiling_on_sc=False,
      ),
      scratch_types=[
          pltpu.VMEM((B_PER_W,), jnp.int32),
          pltpu.VMEM((B_PER_W,), jnp.int32),
          pltpu.VMEM((B_PER_W, EMB_DIM), jnp.float32),
          pltpu.VMEM((B_PER_W, EMB_DIM), jnp.float32),
          pltpu.VMEM((B_PER_W,), jnp.float32),
          pltpu.SemaphoreType.DMA,
      ],
  )
  return run(user, item, user_table, item_table)
